# baseline (speedup 1.0000x reference)
import functools

import jax
import jax.numpy as jnp
from jax import lax
from jax.experimental import pallas as pl
from jax.experimental.pallas import tpu as pltpu
from jax.experimental.pallas import tpu_sc as plsc


def _addend_body(months_ref, ch_ref, pos_ref, mon_ref, out_ref):
    b, t, bs, d = out_ref.shape
    n = ch_ref.shape[1]
    ch = ch_ref[...]
    zero = jnp.zeros((bs, n), jnp.float32)
    for bi in range(b):
        for ti in range(t):
            m = months_ref[bi, ti]
            row_m = mon_ref[pl.ds(m, 1), :]
            row3 = jnp.concatenate([
                ch,
                jnp.broadcast_to(pos_ref[ti:ti + 1, :], (bs, n)),
                jnp.broadcast_to(row_m, (bs, n)),
                zero,
            ], axis=-1)
            out_ref[bi, ti] = row3


def kernel(tokens, timestamps, channel_embed, pos_embed, month_table):
    b, h, w, t, bs, d = tokens.shape
    n = d // 4
    months = timestamps[:, :, 1].astype(jnp.int32)

    a_small = pl.pallas_call(
        _addend_body,
        in_specs=[
            pl.BlockSpec(memory_space=pltpu.SMEM),
            pl.BlockSpec(memory_space=pltpu.VMEM),
            pl.BlockSpec(memory_space=pltpu.VMEM),
            pl.BlockSpec(memory_space=pltpu.VMEM),
        ],
        out_shape=jax.ShapeDtypeStruct((b, t, bs, d), jnp.float32),
    )(months, channel_embed, pos_embed, month_table)

    NC, NS = 2, 16
    NW = NC * NS
    units = b * h * w
    upw = units // NW
    nvec = (3 * n) // 16

    mesh = plsc.VectorSubcoreMesh(core_axis_name="c", subcore_axis_name="s")

    @functools.partial(
        pl.kernel,
        mesh=mesh,
        out_type=jax.ShapeDtypeStruct(tokens.shape, tokens.dtype),
        scratch_types=[
            pltpu.VMEM((t, bs, d), jnp.float32),
            pltpu.VMEM((t, bs, d), jnp.float32),
        ],
    )
    def _sc_add(tok_hbm, a_hbm, out_hbm, buf_v, a_v):
        cid = lax.axis_index("c")
        sid = lax.axis_index("s")
        wid = sid * NC + cid
        b_idx = wid // (NW // b)
        pltpu.sync_copy(a_hbm.at[b_idx], a_v)

        def unit_body(j, carry):
            u = wid * upw + j
            bi = u // (h * w)
            rem = u % (h * w)
            hi = rem // w
            wi = rem % w
            pltpu.sync_copy(tok_hbm.at[bi, hi, wi], buf_v)
            for ti in range(t):
                for bsi in range(bs):
                    def addk(k, c2):
                        sl = pl.ds(k * 16, 16)
                        buf_v[ti, bsi, sl] = buf_v[ti, bsi, sl] + a_v[ti, bsi, sl]
                        return c2
                    lax.fori_loop(0, nvec, addk, 0)
            pltpu.sync_copy(buf_v, out_hbm.at[bi, hi, wi])
            return carry
        lax.fori_loop(0, upw, unit_body, 0)

    return _sc_add(tokens, a_small)


# speedup vs baseline: 1.1588x; 1.1588x over previous
import functools

import jax
import jax.numpy as jnp
from jax import lax
from jax.experimental import pallas as pl
from jax.experimental.pallas import tpu as pltpu
from jax.experimental.pallas import tpu_sc as plsc


def _addend_body(months_ref, ch_ref, pos_ref, mon_ref, out_ref):
    b, t, bs, d = out_ref.shape
    n = ch_ref.shape[1]
    ch = ch_ref[...]
    zero = jnp.zeros((bs, n), jnp.float32)
    for bi in range(b):
        for ti in range(t):
            m = months_ref[bi, ti]
            row_m = mon_ref[pl.ds(m, 1), :]
            row3 = jnp.concatenate([
                ch,
                jnp.broadcast_to(pos_ref[ti:ti + 1, :], (bs, n)),
                jnp.broadcast_to(row_m, (bs, n)),
                zero,
            ], axis=-1)
            out_ref[bi, ti] = row3


def kernel(tokens, timestamps, channel_embed, pos_embed, month_table):
    b, h, w, t, bs, d = tokens.shape
    n = d // 4
    months = timestamps[:, :, 1].astype(jnp.int32)

    a_small = pl.pallas_call(
        _addend_body,
        in_specs=[
            pl.BlockSpec(memory_space=pltpu.SMEM),
            pl.BlockSpec(memory_space=pltpu.VMEM),
            pl.BlockSpec(memory_space=pltpu.VMEM),
            pl.BlockSpec(memory_space=pltpu.VMEM),
        ],
        out_shape=jax.ShapeDtypeStruct((b, t, bs, d), jnp.float32),
    )(months, channel_embed, pos_embed, month_table)

    NC, NS = 2, 16
    NW = NC * NS
    units = b * h * w
    upw = units // NW
    nvec = (3 * n) // 16

    mesh = plsc.VectorSubcoreMesh(core_axis_name="c", subcore_axis_name="s")

    nbuf = 2

    @functools.partial(
        pl.kernel,
        mesh=mesh,
        out_type=jax.ShapeDtypeStruct(tokens.shape, tokens.dtype),
        scratch_types=[
            pltpu.VMEM((nbuf, t, bs, d), jnp.float32),
            pltpu.VMEM((t, bs, d), jnp.float32),
        ]
        + [pltpu.SemaphoreType.DMA] * (2 * nbuf),
    )
    def _sc_add(tok_hbm, a_hbm, out_hbm, buf_v, a_v, *sems):
        sem_in = sems[:nbuf]
        sem_out = sems[nbuf:]
        cid = lax.axis_index("c")
        sid = lax.axis_index("s")
        wid = sid * NC + cid
        b_idx = wid // (NW // b)
        pltpu.sync_copy(a_hbm.at[b_idx], a_v)

        def unit_coords(u):
            bi = u // (h * w)
            rem = u % (h * w)
            return bi, rem // w, rem % w

        def start_in(j, slot):
            bi, hi, wi = unit_coords(wid * upw + j)
            pltpu.make_async_copy(
                tok_hbm.at[bi, hi, wi], buf_v.at[slot], sem_in[slot]).start()

        def start_out(j, slot):
            bi, hi, wi = unit_coords(wid * upw + j)
            pltpu.make_async_copy(
                buf_v.at[slot], out_hbm.at[bi, hi, wi], sem_out[slot]).start()

        def wait_in(j, slot):
            bi, hi, wi = unit_coords(wid * upw + j)
            pltpu.make_async_copy(
                tok_hbm.at[bi, hi, wi], buf_v.at[slot], sem_in[slot]).wait()

        def wait_out(j, slot):
            bi, hi, wi = unit_coords(wid * upw + j)
            pltpu.make_async_copy(
                buf_v.at[slot], out_hbm.at[bi, hi, wi], sem_out[slot]).wait()

        for slot in range(nbuf):
            start_in(slot, slot)

        def outer(j2, carry):
            for slot in range(nbuf):
                j = j2 * nbuf + slot
                wait_in(j, slot)
                for ti in range(t):
                    for bsi in range(bs):
                        def addk(k, c2):
                            sl = pl.ds(k * 16, 16)
                            buf_v[slot, ti, bsi, sl] = (
                                buf_v[slot, ti, bsi, sl] + a_v[ti, bsi, sl])
                            return c2
                        lax.fori_loop(0, nvec, addk, 0)
                start_out(j, slot)
                nxt = j + nbuf

                @pl.when(nxt < upw)
                def _():
                    wait_out(nxt - nbuf, slot)
                    start_in(nxt, slot)
            return carry
        lax.fori_loop(0, upw // nbuf, outer, 0)

        for slot in range(nbuf):
            wait_out(upw - nbuf + slot, slot)

    return _sc_add(tokens, a_small)


# speedup vs baseline: 1.3281x; 1.1461x over previous
import functools

import jax
import jax.numpy as jnp
from jax import lax
from jax.experimental import pallas as pl
from jax.experimental.pallas import tpu as pltpu
from jax.experimental.pallas import tpu_sc as plsc


def _addend_body(months_ref, ch_ref, pos_ref, mon_ref, out_ref):
    b, t, bs, d = out_ref.shape
    n = ch_ref.shape[1]
    ch = ch_ref[...]
    zero = jnp.zeros((bs, n), jnp.float32)
    for bi in range(b):
        for ti in range(t):
            m = months_ref[bi, ti]
            row_m = mon_ref[pl.ds(m, 1), :]
            row3 = jnp.concatenate([
                ch,
                jnp.broadcast_to(pos_ref[ti:ti + 1, :], (bs, n)),
                jnp.broadcast_to(row_m, (bs, n)),
                zero,
            ], axis=-1)
            out_ref[bi, ti] = row3


def kernel(tokens, timestamps, channel_embed, pos_embed, month_table):
    b, h, w, t, bs, d = tokens.shape
    n = d // 4
    months = timestamps[:, :, 1].astype(jnp.int32)

    a_small = pl.pallas_call(
        _addend_body,
        in_specs=[
            pl.BlockSpec(memory_space=pltpu.SMEM),
            pl.BlockSpec(memory_space=pltpu.VMEM),
            pl.BlockSpec(memory_space=pltpu.VMEM),
            pl.BlockSpec(memory_space=pltpu.VMEM),
        ],
        out_shape=jax.ShapeDtypeStruct((b, t, bs, d), jnp.float32),
    )(months, channel_embed, pos_embed, month_table)

    NC, NS = 2, 16
    NW = NC * NS
    units = b * h * w
    upw = units // NW
    nvec = (3 * n) // 16

    mesh = plsc.VectorSubcoreMesh(core_axis_name="c", subcore_axis_name="s")

    nbuf = 2

    @functools.partial(
        pl.kernel,
        mesh=mesh,
        out_type=jax.ShapeDtypeStruct(tokens.shape, tokens.dtype),
        scratch_types=[
            pltpu.VMEM((nbuf, t, bs, d), jnp.float32),
            pltpu.VMEM((t, bs, d), jnp.float32),
        ]
        + [pltpu.SemaphoreType.DMA] * (2 * nbuf),
    )
    def _sc_add(tok_hbm, a_hbm, out_hbm, buf_v, a_v, *sems):
        sem_in = sems[:nbuf]
        sem_out = sems[nbuf:]
        cid = lax.axis_index("c")
        sid = lax.axis_index("s")
        wid = sid * NC + cid
        b_idx = wid // (NW // b)
        pltpu.sync_copy(a_hbm.at[b_idx], a_v)

        def unit_coords(u):
            bi = u // (h * w)
            rem = u % (h * w)
            return bi, rem // w, rem % w

        def start_in(j, slot):
            bi, hi, wi = unit_coords(wid * upw + j)
            pltpu.make_async_copy(
                tok_hbm.at[bi, hi, wi], buf_v.at[slot], sem_in[slot]).start()

        def start_out(j, slot):
            bi, hi, wi = unit_coords(wid * upw + j)
            pltpu.make_async_copy(
                buf_v.at[slot], out_hbm.at[bi, hi, wi], sem_out[slot]).start()

        def wait_in(j, slot):
            bi, hi, wi = unit_coords(wid * upw + j)
            pltpu.make_async_copy(
                tok_hbm.at[bi, hi, wi], buf_v.at[slot], sem_in[slot]).wait()

        def wait_out(j, slot):
            bi, hi, wi = unit_coords(wid * upw + j)
            pltpu.make_async_copy(
                buf_v.at[slot], out_hbm.at[bi, hi, wi], sem_out[slot]).wait()

        for slot in range(nbuf):
            start_in(slot, slot)

        def outer(j2, carry):
            for slot in range(nbuf):
                j = j2 * nbuf + slot
                wait_in(j, slot)

                def addrow(q, c2):
                    ti = q // bs
                    bsi = q % bs
                    for k in range(nvec):
                        sl = pl.ds(k * 16, 16)
                        buf_v[slot, ti, bsi, sl] = (
                            buf_v[slot, ti, bsi, sl] + a_v[ti, bsi, sl])
                    return c2
                lax.fori_loop(0, t * bs, addrow, 0)
                start_out(j, slot)
                nxt = j + nbuf

                @pl.when(nxt < upw)
                def _():
                    wait_out(nxt - nbuf, slot)
                    start_in(nxt, slot)
            return carry
        lax.fori_loop(0, upw // nbuf, outer, 0)

        for slot in range(nbuf):
            wait_out(upw - nbuf + slot, slot)

    return _sc_add(tokens, a_small)
